# Initial kernel scaffold; baseline (speedup 1.0000x reference)
import jax
import jax.numpy as jnp
from jax.experimental import pallas as pl


def kernel(x, edge_index, hidden_states, Wx0_self, Wx0_neigh, bx0, Wx1_self, Wx1_neigh, bx1, Wh_self, Wh_neigh, bh):
    raise NotImplementedError("write your pallas kernel here")


# speedup vs baseline: 2.3203x; 2.3203x over previous
import functools

import jax
import jax.numpy as jnp
from jax import lax
from jax.experimental import pallas as pl
from jax.experimental.pallas import tpu as pltpu
from jax.experimental.pallas import tpu_sc as plsc

N = 10000
E = 320000
SEQ = 8
L = 2
DIN = 128
DOUT = 64

NUM_CORES = 2
NUM_SUBCORES = 16
NUM_TILES = NUM_CORES * NUM_SUBCORES

CH = 128
CHUNKS = (E + NUM_TILES * CH - 1) // (NUM_TILES * CH)
EDGES_PER_TILE = CHUNKS * CH
E_PAD = NUM_TILES * EDGES_PER_TILE

N_ACC = 10112
ROWS_PER_TILE = N_ACC // NUM_SUBCORES
ZB = ROWS_PER_TILE // 4


@functools.lru_cache(maxsize=None)
def _make_seg_sum(d):
    mesh = plsc.VectorSubcoreMesh(core_axis_name="c", subcore_axis_name="s")

    @functools.partial(
        pl.kernel,
        out_type=jax.ShapeDtypeStruct((NUM_CORES, N_ACC, d), jnp.float32),
        mesh=mesh,
        compiler_params=pltpu.CompilerParams(use_tc_tiling_on_sc=False),
        scratch_types=[
            pltpu.VMEM((CH,), jnp.int32),
            pltpu.VMEM((CH,), jnp.int32),
            pltpu.VMEM((CH, d), jnp.float32),
            pltpu.VMEM((ZB, d), jnp.float32),
            pltpu.VMEM_SHARED((N_ACC, d), jnp.float32),
            pltpu.SemaphoreType.DMA,
        ],
    )
    def seg_sum(feat_hbm, src_hbm, dst_hbm, out_hbm, src_v, dst_v, rows_v,
                zeros_v, acc_sh, sem):
        cid = lax.axis_index("c")
        sid = lax.axis_index("s")
        wid = cid * NUM_SUBCORES + sid

        zvec = jnp.zeros((16,), jnp.float32)

        def zero_row(r, _):
            for j in range(d // 16):
                zeros_v[r, pl.ds(j * 16, 16)] = zvec
            return 0

        lax.fori_loop(0, ZB, zero_row, 0)
        stripe = sid * ROWS_PER_TILE
        for q in range(4):
            pltpu.sync_copy(zeros_v, acc_sh.at[pl.ds(stripe + q * ZB, ZB)])
        plsc.subcore_barrier()

        base = wid * EDGES_PER_TILE

        def body(i, _):
            off = base + i * CH
            pltpu.sync_copy(src_hbm.at[pl.ds(off, CH)], src_v)
            pltpu.sync_copy(dst_hbm.at[pl.ds(off, CH)], dst_v)
            pltpu.async_copy(feat_hbm.at[src_v], rows_v, sem).wait()
            pltpu.sync_copy(rows_v, acc_sh.at[dst_v], add=True)
            return 0

        lax.fori_loop(0, CHUNKS, body, 0)
        plsc.subcore_barrier()

        pltpu.sync_copy(acc_sh.at[pl.ds(stripe, ROWS_PER_TILE)],
                        out_hbm.at[cid, pl.ds(stripe, ROWS_PER_TILE)])

    return seg_sum


def _cell_body(xin_ref, ax_ref, h_ref, ah_ref, deg_ref, wxs_ref, wxn_ref,
               whs_ref, whn_ref, b_ref, out_ref):
    inv = 1.0 / jnp.maximum(deg_ref[0, :, 0:1] + deg_ref[1, :, 0:1], 1.0)
    mx = (ax_ref[0] + ax_ref[1]) * inv
    mh = (ah_ref[0] + ah_ref[1]) * inv
    h = h_ref[...]
    pre = (jnp.dot(xin_ref[...], wxs_ref[...], preferred_element_type=jnp.float32)
           + jnp.dot(mx, wxn_ref[...], preferred_element_type=jnp.float32)
           + jnp.dot(h, whs_ref[...], preferred_element_type=jnp.float32)
           + jnp.dot(mh, whn_ref[...], preferred_element_type=jnp.float32)
           + b_ref[...])
    u = jax.nn.sigmoid(pre[:, :DOUT])
    c = jnp.tanh(pre[:, DOUT:])
    out_ref[...] = u * h + (1.0 - u) * c


@functools.lru_cache(maxsize=None)
def _make_cell(din):
    BLK = 1000
    grid = (N // BLK,)
    w2 = 2 * DOUT

    def rows(i):
        return (i, 0)

    def rows3(i):
        return (0, i, 0)

    def full2(i):
        return (0, 0)

    return pl.pallas_call(
        _cell_body,
        grid=grid,
        in_specs=[
            pl.BlockSpec((BLK, din), rows),
            pl.BlockSpec((NUM_CORES, BLK, din), rows3),
            pl.BlockSpec((BLK, DOUT), rows),
            pl.BlockSpec((NUM_CORES, BLK, DOUT), rows3),
            pl.BlockSpec((NUM_CORES, BLK, 16), rows3),
            pl.BlockSpec((din, w2), full2),
            pl.BlockSpec((din, w2), full2),
            pl.BlockSpec((DOUT, w2), full2),
            pl.BlockSpec((DOUT, w2), full2),
            pl.BlockSpec((1, w2), full2),
        ],
        out_specs=pl.BlockSpec((BLK, DOUT), rows),
        out_shape=jax.ShapeDtypeStruct((N, DOUT), jnp.float32),
    )


def kernel(x, edge_index, hidden_states, Wx0_self, Wx0_neigh, bx0,
           Wx1_self, Wx1_neigh, bx1, Wh_self, Wh_neigh, bh):
    src = edge_index[0]
    dst = edge_index[1]
    pad = E_PAD - E
    srcp = jnp.concatenate([src, jnp.zeros((pad,), jnp.int32)])
    dstp = jnp.concatenate([dst, jnp.full((pad,), N, jnp.int32)])

    def cat(w):
        return jnp.concatenate([w[1], w[2]], axis=-1)

    wx_s = [cat(Wx0_self), cat(Wx1_self)]
    wx_n = [cat(Wx0_neigh), cat(Wx1_neigh)]
    wh_s = [cat(Wh_self[l]) for l in range(L)]
    wh_n = [cat(Wh_neigh[l]) for l in range(L)]
    bias = [(cat(bx0[:, None, :])[0] + cat(bh[0][:, None, :])[0])[None, :],
            (cat(bx1[:, None, :])[0] + cat(bh[1][:, None, :])[0])[None, :]]

    seg64 = _make_seg_sum(DOUT)
    seg128 = _make_seg_sum(DIN)
    seg16 = _make_seg_sum(16)
    cell0 = _make_cell(DIN)
    cell1 = _make_cell(DOUT)
    cells = [cell0, cell1]

    ones = jnp.ones((N, 16), jnp.float32)
    deg_p = seg16(ones, srcp, dstp)

    aggx = [seg128(x[i], srcp, dstp) for i in range(SEQ)]
    h0 = hidden_states[0]
    h1 = hidden_states[1]
    aggh0 = seg64(h0, srcp, dstp)
    aggh1 = seg64(h1, srcp, dstp)

    for i in range(SEQ):
        out0 = cell0(x[i], aggx[i], h0, aggh0, deg_p,
                     wx_s[0], wx_n[0], wh_s[0], wh_n[0], bias[0])
        agg_out0 = seg64(out0, srcp, dstp)
        out1 = cell1(out0, agg_out0, h1, aggh1, deg_p,
                     wx_s[1], wx_n[1], wh_s[1], wh_n[1], bias[1])
        h0, aggh0 = out0, agg_out0
        h1 = out1
        if i < SEQ - 1:
            aggh1 = seg64(out1, srcp, dstp)

    return (x, jnp.stack([h0, h1], axis=0))
